# baseline (speedup 1.0000x reference)
import functools

import jax
import jax.numpy as jnp
from jax import lax
from jax.experimental import pallas as pl
from jax.experimental.pallas import tpu as pltpu
from jax.experimental.pallas import tpu_sc as plsc

_NC = 2
_NS = 16
_NW = _NC * _NS
_C = 16


def kernel(input_ids, embed_table):
    b, s = input_ids.shape
    d = embed_table.shape[1]
    per_w = (b * s) // _NW
    nchunk = per_w // _C
    w_per_b = s // per_w
    mesh = plsc.VectorSubcoreMesh(
        core_axis_name="c", subcore_axis_name="s",
        num_cores=_NC, num_subcores=_NS)

    @functools.partial(
        pl.kernel,
        out_type=jax.ShapeDtypeStruct((b, s, d), jnp.float32),
        mesh=mesh,
        scratch_types=[
            pltpu.VMEM((per_w,), jnp.int32),
            pltpu.VMEM((_C, d), jnp.float32),
            pltpu.VMEM((_C, d), jnp.float32),
            pltpu.VMEM((_C, d), jnp.float32),
            pltpu.VMEM((_C, d), jnp.float32),
            pltpu.VMEM((_C, d), jnp.float32),
            pltpu.VMEM((_C, d), jnp.float32),
            pltpu.VMEM((_C, d), jnp.float32),
            pltpu.SemaphoreType.DMA,
            pltpu.SemaphoreType.DMA,
            pltpu.SemaphoreType.DMA,
            pltpu.SemaphoreType.DMA,
            pltpu.SemaphoreType.DMA,
            pltpu.SemaphoreType.DMA,
            pltpu.SemaphoreType.DMA,
            pltpu.SemaphoreType.DMA,
            pltpu.SemaphoreType.DMA,
            pltpu.SemaphoreType.DMA,
            pltpu.SemaphoreType.DMA,
            pltpu.SemaphoreType.DMA,
            pltpu.SemaphoreType.DMA,
            pltpu.SemaphoreType.DMA,
        ],
    )
    def k(ids_hbm, table_hbm, out_hbm, idx_v,
          buf0, buf1, buf2, buf3, buf4, buf5, buf6,
          gs0, gs1, gs2, gs3, gs4, gs5, gs6,
          ws0, ws1, ws2, ws3, ws4, ws5, ws6):
        wid = lax.axis_index("s") * _NC + lax.axis_index("c")
        row = wid // w_per_b
        off = (wid % w_per_b) * per_w
        pltpu.sync_copy(ids_hbm.at[row, pl.ds(off, per_w)], idx_v)
        nbuf = 7
        bufs = (buf0, buf1, buf2, buf3, buf4, buf5, buf6)
        gsems = (gs0, gs1, gs2, gs3, gs4, gs5, gs6)
        wsems = (ws0, ws1, ws2, ws3, ws4, ws5, ws6)
        h = _C // 2
        def gather(j, buf, sem):
            a = pltpu.async_copy(
                table_hbm.at[idx_v.at[pl.ds(j * _C, h)]],
                buf.at[pl.ds(0, h)], sem)
            bcp = pltpu.async_copy(
                table_hbm.at[idx_v.at[pl.ds(j * _C + h, h)]],
                buf.at[pl.ds(h, h)], sem)
            return (a, bcp)
        gcps = [None] * nbuf
        wcps = [None] * nbuf
        for j in range(nbuf):
            gcps[j] = gather(j, bufs[j], gsems[j])
        for j in range(nchunk):
            cur = j % nbuf
            gcps[cur][0].wait()
            gcps[cur][1].wait()
            wcps[cur] = pltpu.async_copy(
                bufs[cur], out_hbm.at[row, pl.ds(off + j * _C, _C)],
                wsems[cur])
            nj = j + nbuf
            if nj < nchunk:
                wcps[cur].wait()
                gcps[cur] = gather(nj, bufs[cur], gsems[cur])
        for j in range(nchunk - nbuf, nchunk):
            wcps[j % nbuf].wait()

    return k(input_ids.astype(jnp.int32), embed_table)


# speedup vs baseline: 1.0132x; 1.0132x over previous
import functools

import jax
import jax.numpy as jnp
from jax import lax
from jax.experimental import pallas as pl
from jax.experimental.pallas import tpu as pltpu
from jax.experimental.pallas import tpu_sc as plsc

_NC = 2
_NS = 16
_NW = _NC * _NS
_C = 16


def kernel(input_ids, embed_table):
    b, s = input_ids.shape
    d = embed_table.shape[1]
    per_w = (b * s) // _NW
    nchunk = per_w // _C
    w_per_b = s // per_w
    mesh = plsc.VectorSubcoreMesh(
        core_axis_name="c", subcore_axis_name="s",
        num_cores=_NC, num_subcores=_NS)

    @functools.partial(
        pl.kernel,
        out_type=jax.ShapeDtypeStruct((b, s, d), jnp.float32),
        mesh=mesh,
        scratch_types=[
            pltpu.VMEM((per_w,), jnp.int32),
            pltpu.VMEM((_C, d), jnp.float32),
            pltpu.VMEM((_C, d), jnp.float32),
            pltpu.VMEM((_C, d), jnp.float32),
            pltpu.VMEM((_C, d), jnp.float32),
            pltpu.VMEM((_C, d), jnp.float32),
            pltpu.VMEM((_C, d), jnp.float32),
            pltpu.VMEM((_C, d), jnp.float32),
            pltpu.SemaphoreType.DMA,
            pltpu.SemaphoreType.DMA,
            pltpu.SemaphoreType.DMA,
            pltpu.SemaphoreType.DMA,
            pltpu.SemaphoreType.DMA,
            pltpu.SemaphoreType.DMA,
            pltpu.SemaphoreType.DMA,
            pltpu.SemaphoreType.DMA,
            pltpu.SemaphoreType.DMA,
            pltpu.SemaphoreType.DMA,
            pltpu.SemaphoreType.DMA,
            pltpu.SemaphoreType.DMA,
            pltpu.SemaphoreType.DMA,
            pltpu.SemaphoreType.DMA,
        ],
    )
    def k(ids_hbm, table_hbm, out_hbm, idx_v,
          buf0, buf1, buf2, buf3, buf4, buf5, buf6,
          gs0, gs1, gs2, gs3, gs4, gs5, gs6,
          ws0, ws1, ws2, ws3, ws4, ws5, ws6):
        wid = lax.axis_index("s") * _NC + lax.axis_index("c")
        row = wid // w_per_b
        off = (wid % w_per_b) * per_w
        pltpu.sync_copy(ids_hbm.at[row, pl.ds(off, per_w)], idx_v)
        nbuf = 7
        bufs = (buf0, buf1, buf2, buf3, buf4, buf5, buf6)
        gsems = (gs0, gs1, gs2, gs3, gs4, gs5, gs6)
        wsems = (ws0, ws1, ws2, ws3, ws4, ws5, ws6)
        gcps = [None] * nbuf
        wcps = [None] * nbuf
        for j in range(nbuf):
            gcps[j] = pltpu.async_copy(
                table_hbm.at[idx_v.at[pl.ds(j * _C, _C)]], bufs[j], gsems[j])
        for j in range(nchunk):
            cur = j % nbuf
            gcps[cur].wait()
            wcps[cur] = pltpu.async_copy(
                bufs[cur], out_hbm.at[row, pl.ds(off + j * _C, _C)],
                wsems[cur])
            nj = j + nbuf
            if nj < nchunk:
                wcps[cur].wait()
                gcps[cur] = pltpu.async_copy(
                    table_hbm.at[idx_v.at[pl.ds(nj * _C, _C)]],
                    bufs[cur], gsems[cur])
        for j in range(nchunk - nbuf, nchunk):
            wcps[j % nbuf].wait()

    return k(input_ids.astype(jnp.int32), embed_table)
